# baseline (speedup 1.0000x reference)
import functools

import jax
import jax.numpy as jnp
from jax import lax
from jax.experimental import pallas as pl
from jax.experimental.pallas import tpu as pltpu
from jax.experimental.pallas import tpu_sc as plsc

B = 16384
DIM = 64
OUT_DIM = 4 * DIM
TEMPO_MIN = 60.0
TEMPO_MAX = 200.0
TEMPO_BINS = 100
DUR_MAX = 120.0
MOOD_V = 1000

NC = 2
NS = 16
NW = NC * NS
BPW = B // NW
CH = 128
HALF = BPW // 4
NH = BPW // HALF
LANES = 16
PDIM = DIM + 1


def _sc_gather_body(gids, mids, bins_hbm, gt, mt, tt,
                    cat_out,
                    gidx, midx, tbin, grow, mrow, trow,
                    mood_v, tempo_v, mood_s, tempo_s, sem):
    sid = lax.axis_index("s")
    wid = sid * NC + lax.axis_index("c")
    base = wid * BPW

    @pl.when(sid == 0)
    def _():
        pltpu.sync_copy(mt, mood_s)
        pltpu.sync_copy(tt, tempo_s)

    plsc.subcore_barrier()
    pltpu.sync_copy(mood_s, mood_v.at[:, pl.ds(0, DIM)])
    pltpu.sync_copy(tempo_s, tempo_v.at[:, pl.ds(0, DIM)])

    pltpu.sync_copy(gids.at[pl.ds(base, BPW)], gidx)
    pltpu.sync_copy(mids.at[pl.ds(base, BPW)], midx)
    pltpu.sync_copy(bins_hbm.at[pl.ds(base, BPW)], tbin)

    row_iota = jax.lax.iota(jnp.int32, LANES)
    out_handles = []

    for h in range(NH):
        for c in out_handles:
            c.wait()
        out_handles = []

        g_handles = []
        for i in range(HALF // CH):
            idx_slice = gidx.at[pl.ds(h * HALF + i * CH, CH)]
            g_handles.append(
                pltpu.async_copy(gt.at[idx_slice], grow.at[pl.ds(i * CH, CH)], sem))

        def group_body(g, carry):
            mids16 = midx[pl.ds(h * HALF + g * LANES, LANES)]
            tids16 = tbin[pl.ds(h * HALF + g * LANES, LANES)]
            rows16 = row_iota + g * LANES
            for f in range(DIM):
                fv = jnp.full((LANES,), f, jnp.int32)
                mv = plsc.load_gather(mood_v, [mids16, fv])
                plsc.store_scatter(mrow, [rows16, fv], mv)
                tv = plsc.load_gather(tempo_v, [tids16, fv])
                plsc.store_scatter(trow, [rows16, fv], tv)
            return carry

        lax.fori_loop(0, HALF // LANES, group_body, 0)

        for c in g_handles:
            c.wait()

        hb = base + h * HALF
        out_handles.append(pltpu.async_copy(
            grow, cat_out.at[pl.ds(hb, HALF), pl.ds(0, DIM)], sem))
        out_handles.append(pltpu.async_copy(
            mrow.at[:, pl.ds(0, DIM)],
            cat_out.at[pl.ds(hb, HALF), pl.ds(DIM, DIM)], sem))
        out_handles.append(pltpu.async_copy(
            trow.at[:, pl.ds(0, DIM)],
            cat_out.at[pl.ds(hb, HALF), pl.ds(2 * DIM, DIM)], sem))

    for c in out_handles:
        c.wait()


@jax.jit
def _sc_gather(gids, mids, bins, gt, mt, tt):
    mesh = plsc.VectorSubcoreMesh(core_axis_name="c", subcore_axis_name="s")
    f = functools.partial(
        pl.kernel,
        mesh=mesh,
        compiler_params=pltpu.CompilerParams(
            use_tc_tiling_on_sc=False, needs_layout_passes=False),
        out_type=jax.ShapeDtypeStruct((B, OUT_DIM), jnp.float32),
        scratch_types=[
            pltpu.VMEM((BPW,), jnp.int32),
            pltpu.VMEM((BPW,), jnp.int32),
            pltpu.VMEM((BPW,), jnp.int32),
            pltpu.VMEM((HALF, DIM), jnp.float32),
            pltpu.VMEM((HALF, PDIM), jnp.float32),
            pltpu.VMEM((HALF, PDIM), jnp.float32),
            pltpu.VMEM((MOOD_V, PDIM), jnp.float32),
            pltpu.VMEM((TEMPO_BINS, PDIM), jnp.float32),
            pltpu.VMEM_SHARED((MOOD_V, DIM), jnp.float32),
            pltpu.VMEM_SHARED((TEMPO_BINS, DIM), jnp.float32),
            pltpu.SemaphoreType.DMA,
        ],
    )(_sc_gather_body)
    return f(gids, mids, bins, gt, mt, tt)


def _tc_fuse_body(cat, d, w, bias, gam, bet, o):
    dn = jnp.clip(d[:] / DUR_MAX, 0.0, 1.0)
    dur = dn * w[:] + bias[:]
    cond = jnp.concatenate([cat[:, :3 * DIM], dur], axis=1)
    mu = jnp.mean(cond, axis=1, keepdims=True)
    var = jnp.mean(jnp.square(cond - mu), axis=1, keepdims=True)
    o[:] = (cond - mu) * lax.rsqrt(var + 1e-5) * gam[:] + bet[:]


@jax.jit
def _tc_fuse(cat, dur, w_row, b_row, gamma_row, beta_row):
    BLK = 512
    grid = (B // BLK,)
    vec_spec = pl.BlockSpec((1, DIM), lambda i: (0, 0))
    out_spec = pl.BlockSpec((BLK, OUT_DIM), lambda i: (i, 0))
    return pl.pallas_call(
        _tc_fuse_body,
        grid=grid,
        in_specs=[
            pl.BlockSpec((BLK, OUT_DIM), lambda i: (i, 0)),
            pl.BlockSpec((BLK, 1), lambda i: (i, 0)),
            vec_spec, vec_spec,
            pl.BlockSpec((1, OUT_DIM), lambda i: (0, 0)),
            pl.BlockSpec((1, OUT_DIM), lambda i: (0, 0)),
        ],
        out_specs=out_spec,
        out_shape=jax.ShapeDtypeStruct((B, OUT_DIM), jnp.float32),
    )(cat, dur, w_row, b_row, gamma_row, beta_row)


def kernel(genre_ids, mood_ids, tempo, duration, genre_table, mood_table,
           tempo_table, dur_W, dur_b, gamma, beta):
    t = jnp.clip(tempo, TEMPO_MIN, TEMPO_MAX)
    normalized = (t - TEMPO_MIN) / (TEMPO_MAX - TEMPO_MIN)
    bins = (normalized * (TEMPO_BINS - 1)).astype(jnp.int32)
    cat = _sc_gather(genre_ids, mood_ids, bins,
                     genre_table, mood_table, tempo_table)
    return _tc_fuse(cat,
                    duration.reshape(B, 1),
                    dur_W.reshape(1, DIM),
                    dur_b.reshape(1, DIM),
                    gamma.reshape(1, OUT_DIM),
                    beta.reshape(1, OUT_DIM))


# speedup vs baseline: 1.0816x; 1.0816x over previous
import functools

import jax
import jax.numpy as jnp
from jax import lax
from jax.experimental import pallas as pl
from jax.experimental.pallas import tpu as pltpu
from jax.experimental.pallas import tpu_sc as plsc

B = 16384
DIM = 64
OUT_DIM = 4 * DIM
TEMPO_MIN = 60.0
TEMPO_MAX = 200.0
TEMPO_BINS = 100
DUR_MAX = 120.0
MOOD_V = 1000

NC = 2
NS = 16
NW = NC * NS
BPW = B // NW
CH = 128
HALF = BPW // 4
NH = BPW // HALF
LANES = 16
PDIM = DIM + 1


def _sc_gather_body(gids, mids, bins_hbm, gt, mt, tt,
                    cat_out,
                    gidx, midx, tbin, grow, mrow, trow,
                    mood_v, tempo_v, mood_s, tempo_s, sem):
    sid = lax.axis_index("s")
    wid = sid * NC + lax.axis_index("c")
    base = wid * BPW

    @pl.when(sid == 0)
    def _():
        pltpu.sync_copy(mt, mood_s)
        pltpu.sync_copy(tt, tempo_s)

    plsc.subcore_barrier()
    pltpu.sync_copy(mood_s, mood_v.at[:, pl.ds(0, DIM)])
    pltpu.sync_copy(tempo_s, tempo_v.at[:, pl.ds(0, DIM)])

    pltpu.sync_copy(gids.at[pl.ds(base, BPW)], gidx)
    pltpu.sync_copy(mids.at[pl.ds(base, BPW)], midx)
    pltpu.sync_copy(bins_hbm.at[pl.ds(base, BPW)], tbin)

    row_iota = jax.lax.iota(jnp.int32, LANES)
    out_handles = []

    for h in range(NH):
        for c in out_handles:
            c.wait()
        out_handles = []

        g_handles = []
        for i in range(HALF // CH):
            idx_slice = gidx.at[pl.ds(h * HALF + i * CH, CH)]
            g_handles.append(
                pltpu.async_copy(gt.at[idx_slice], grow.at[pl.ds(i * CH, CH)], sem))

        def group_body(g, carry):
            mids16 = midx[pl.ds(h * HALF + g * LANES, LANES)]
            tids16 = tbin[pl.ds(h * HALF + g * LANES, LANES)]
            rows16 = row_iota + g * LANES
            for f in range(DIM):
                fv = jnp.full((LANES,), f, jnp.int32)
                mv = plsc.load_gather(mood_v, [mids16, fv])
                plsc.store_scatter(mrow, [rows16, fv], mv)
                tv = plsc.load_gather(tempo_v, [tids16, fv])
                plsc.store_scatter(trow, [rows16, fv], tv)
            return carry

        lax.fori_loop(0, HALF // LANES, group_body, 0)

        for c in g_handles:
            c.wait()

        hb = base + h * HALF
        out_handles.append(pltpu.async_copy(
            grow, cat_out.at[pl.ds(hb, HALF), pl.ds(0, DIM)], sem))
        out_handles.append(pltpu.async_copy(
            mrow.at[:, pl.ds(0, DIM)],
            cat_out.at[pl.ds(hb, HALF), pl.ds(DIM, DIM)], sem))
        out_handles.append(pltpu.async_copy(
            trow.at[:, pl.ds(0, DIM)],
            cat_out.at[pl.ds(hb, HALF), pl.ds(2 * DIM, DIM)], sem))

    for c in out_handles:
        c.wait()


@jax.jit
def _sc_gather(gids, mids, bins, gt, mt, tt):
    mesh = plsc.VectorSubcoreMesh(core_axis_name="c", subcore_axis_name="s")
    f = functools.partial(
        pl.kernel,
        mesh=mesh,
        compiler_params=pltpu.CompilerParams(
            use_tc_tiling_on_sc=False, needs_layout_passes=False),
        out_type=jax.ShapeDtypeStruct((B, OUT_DIM), jnp.float32),
        scratch_types=[
            pltpu.VMEM((BPW,), jnp.int32),
            pltpu.VMEM((BPW,), jnp.int32),
            pltpu.VMEM((BPW,), jnp.int32),
            pltpu.VMEM((HALF, DIM), jnp.float32),
            pltpu.VMEM((HALF, PDIM), jnp.float32),
            pltpu.VMEM((HALF, PDIM), jnp.float32),
            pltpu.VMEM((MOOD_V, PDIM), jnp.float32),
            pltpu.VMEM((TEMPO_BINS, PDIM), jnp.float32),
            pltpu.VMEM_SHARED((MOOD_V, DIM), jnp.float32),
            pltpu.VMEM_SHARED((TEMPO_BINS, DIM), jnp.float32),
            pltpu.SemaphoreType.DMA,
        ],
    )(_sc_gather_body)
    return f(gids, mids, bins, gt, mt, tt)


def _tc_fuse_body(cat, d, w, bias, gam, bet, o):
    dn = jnp.clip(d[:] / DUR_MAX, 0.0, 1.0)
    dur = dn * w[:] + bias[:]
    cond = jnp.concatenate([cat[:, :3 * DIM], dur], axis=1)
    mu = jnp.mean(cond, axis=1, keepdims=True)
    var = jnp.mean(jnp.square(cond - mu), axis=1, keepdims=True)
    o[:] = (cond - mu) * lax.rsqrt(var + 1e-5) * gam[:] + bet[:]


@jax.jit
def _tc_fuse(cat, dur, w_row, b_row, gamma_row, beta_row):
    BLK = 4096
    grid = (B // BLK,)
    vec_spec = pl.BlockSpec((1, DIM), lambda i: (0, 0))
    out_spec = pl.BlockSpec((BLK, OUT_DIM), lambda i: (i, 0))
    return pl.pallas_call(
        _tc_fuse_body,
        grid=grid,
        in_specs=[
            pl.BlockSpec((BLK, OUT_DIM), lambda i: (i, 0)),
            pl.BlockSpec((BLK, 1), lambda i: (i, 0)),
            vec_spec, vec_spec,
            pl.BlockSpec((1, OUT_DIM), lambda i: (0, 0)),
            pl.BlockSpec((1, OUT_DIM), lambda i: (0, 0)),
        ],
        out_specs=out_spec,
        out_shape=jax.ShapeDtypeStruct((B, OUT_DIM), jnp.float32),
    )(cat, dur, w_row, b_row, gamma_row, beta_row)


def kernel(genre_ids, mood_ids, tempo, duration, genre_table, mood_table,
           tempo_table, dur_W, dur_b, gamma, beta):
    t = jnp.clip(tempo, TEMPO_MIN, TEMPO_MAX)
    normalized = (t - TEMPO_MIN) / (TEMPO_MAX - TEMPO_MIN)
    bins = (normalized * (TEMPO_BINS - 1)).astype(jnp.int32)
    cat = _sc_gather(genre_ids, mood_ids, bins,
                     genre_table, mood_table, tempo_table)
    return _tc_fuse(cat,
                    duration.reshape(B, 1),
                    dur_W.reshape(1, DIM),
                    dur_b.reshape(1, DIM),
                    gamma.reshape(1, OUT_DIM),
                    beta.reshape(1, OUT_DIM))
